# baseline (speedup 1.0000x reference)
import math

import jax
import jax.numpy as jnp
from jax.experimental import pallas as pl
from jax.experimental.pallas import tpu as pltpu

C_SPARSITY = 0.8


def _pool_body(x_ref, sum_ref, max_ref):
    xb = x_ref[...]
    s = jnp.sum(xb, axis=(3, 4))
    m = jnp.max(xb, axis=(3, 4))
    sum_ref[0, 0, 0, :] = jnp.sum(s, axis=(0, 1))
    max_ref[0, 0, 0, :] = jnp.max(m, axis=(0, 1))


def _make_mask_body(n_pool, k):
    def _mask_body(sum_ref, max_ref, w1_ref, w2_ref, coef_ref):
        avg = sum_ref[...] * (1.0 / n_pool)
        mx = max_ref[...]
        w1 = w1_ref[...]
        w2 = w2_ref[...]

        def mlp(v):
            h = jax.lax.dot_general(v, w1, (((1,), (1,)), ((), ())),
                                    preferred_element_type=jnp.float32)
            h = jnp.maximum(h, 0.0)
            return jax.lax.dot_general(h, w2, (((1,), (1,)), ((), ())),
                                       preferred_element_type=jnp.float32)

        s = jax.nn.sigmoid(mlp(avg) + mlp(mx))
        c = s.shape[1]
        si = s[:, :, None]
        sj = s[:, None, :]
        gt = jnp.sum(jnp.where(sj > si, 1.0, 0.0), axis=2)
        ii = jax.lax.broadcasted_iota(jnp.int32, (1, c, c), 1)
        jj = jax.lax.broadcasted_iota(jnp.int32, (1, c, c), 2)
        eq_lower = jnp.sum(
            jnp.where((sj == si) & (jj < ii), 1.0, 0.0), axis=2)
        rank = gt + eq_lower
        mask = jnp.where(rank < float(k), 1.0, 0.0)
        coef_ref[...] = s * mask
    return _mask_body


def _make_mul_body(cb, hw):
    h, w = hw

    def _mul_body(x_hbm, coef_ref, o_ref, buf, sems):
        b = pl.program_id(0)
        f = pl.program_id(1)
        i = pl.program_id(2)
        nf = pl.num_programs(1)
        ni = pl.num_programs(2)
        n = (b * nf + f) * ni + i
        total = pl.num_programs(0) * nf * ni

        def issue(m):
            bm = m // (nf * ni)
            fm = (m // ni) % nf
            im = m % ni
            slot = m % 2
            for j in range(cb):
                co = coef_ref[bm, im * cb + j]

                @pl.when(co != 0.0)
                def _():
                    pltpu.make_async_copy(
                        x_hbm.at[bm, fm, im * cb + j],
                        buf.at[slot, j],
                        sems.at[slot, j],
                    ).start()

        @pl.when(n == 0)
        def _():
            issue(n)

        @pl.when(n + 1 < total)
        def _():
            issue(n + 1)

        slot = n % 2
        for j in range(cb):
            co = coef_ref[b, i * cb + j]

            @pl.when(co != 0.0)
            def _():
                pltpu.make_async_copy(
                    x_hbm.at[b, f, i * cb + j],
                    buf.at[slot, j],
                    sems.at[slot, j],
                ).wait()
                o_ref[0, 0, j] = buf[slot, j] * co

            @pl.when(co == 0.0)
            def _():
                o_ref[0, 0, j] = jnp.zeros((h, w), jnp.float32)
    return _mul_body


def kernel(x, W1, W2):
    B, F, C, H, W = x.shape
    k = int(math.ceil(C * C_SPARSITY))

    HW = H * W
    LN = 128
    SL = HW // LN
    xv = x.reshape(B, F, C, SL, LN)

    CB1 = 16
    NC1 = C // CB1
    sums, maxs = pl.pallas_call(
        _pool_body,
        grid=(B, NC1),
        in_specs=[pl.BlockSpec((1, F, CB1, SL, LN),
                               lambda b, i: (b, 0, i, 0, 0))],
        out_specs=[pl.BlockSpec((1, 1, 1, CB1), lambda b, i: (b, i, 0, 0)),
                   pl.BlockSpec((1, 1, 1, CB1), lambda b, i: (b, i, 0, 0))],
        out_shape=[jax.ShapeDtypeStruct((B, NC1, 1, CB1), jnp.float32),
                   jax.ShapeDtypeStruct((B, NC1, 1, CB1), jnp.float32)],
    )(xv)
    sums = sums.reshape(B, C)
    maxs = maxs.reshape(B, C)

    coef = pl.pallas_call(
        _make_mask_body(float(F * H * W), k),
        out_shape=jax.ShapeDtypeStruct((B, C), jnp.float32),
    )(sums, maxs, W1, W2)

    CB3 = 16
    out = pl.pallas_call(
        _make_mul_body(CB3, (SL, LN)),
        grid=(B, F, C // CB3),
        in_specs=[pl.BlockSpec(memory_space=pltpu.MemorySpace.HBM),
                  pl.BlockSpec(memory_space=pltpu.SMEM)],
        out_specs=pl.BlockSpec((1, 1, CB3, SL, LN),
                               lambda b, f, i: (b, f, i, 0, 0)),
        out_shape=jax.ShapeDtypeStruct(xv.shape, xv.dtype),
        scratch_shapes=[pltpu.VMEM((2, CB3, SL, LN), jnp.float32),
                        pltpu.SemaphoreType.DMA((2, CB3))],
    )(xv, coef)
    return out.reshape(x.shape)


# speedup vs baseline: 3.3594x; 3.3594x over previous
import math

import jax
import jax.numpy as jnp
from jax.experimental import pallas as pl
from jax.experimental.pallas import tpu as pltpu

C_SPARSITY = 0.8


def _pool_body(x_ref, sum_ref, max_ref):
    xb = x_ref[...]
    s = jnp.sum(xb, axis=(3, 4))
    m = jnp.max(xb, axis=(3, 4))
    sum_ref[0, 0, 0, :] = jnp.sum(s, axis=(0, 1))
    max_ref[0, 0, 0, :] = jnp.max(m, axis=(0, 1))


def _make_mask_body(n_pool, k):
    def _mask_body(sum_ref, max_ref, w1_ref, w2_ref, coef_ref):
        avg = sum_ref[...] * (1.0 / n_pool)
        mx = max_ref[...]
        w1 = w1_ref[...]
        w2 = w2_ref[...]

        def mlp(v):
            h = jax.lax.dot_general(v, w1, (((1,), (1,)), ((), ())),
                                    preferred_element_type=jnp.float32)
            h = jnp.maximum(h, 0.0)
            return jax.lax.dot_general(h, w2, (((1,), (1,)), ((), ())),
                                       preferred_element_type=jnp.float32)

        s = jax.nn.sigmoid(mlp(avg) + mlp(mx))
        c = s.shape[1]
        si = s[:, :, None]
        sj = s[:, None, :]
        gt = jnp.sum(jnp.where(sj > si, 1.0, 0.0), axis=2)
        ii = jax.lax.broadcasted_iota(jnp.int32, (1, c, c), 1)
        jj = jax.lax.broadcasted_iota(jnp.int32, (1, c, c), 2)
        eq_lower = jnp.sum(
            jnp.where((sj == si) & (jj < ii), 1.0, 0.0), axis=2)
        rank = gt + eq_lower
        mask = jnp.where(rank < float(k), 1.0, 0.0)
        coef_ref[...] = s * mask
    return _mask_body


def _make_mul_body(cb, hw):
    h, w = hw

    def _mul_body(x_hbm, coef_ref, o_ref, buf, sems):
        b = pl.program_id(0)
        f = pl.program_id(1)
        i = pl.program_id(2)
        nf = pl.num_programs(1)
        ni = pl.num_programs(2)
        n = (b * nf + f) * ni + i
        total = pl.num_programs(0) * nf * ni

        def issue(m):
            bm = m // (nf * ni)
            fm = (m // ni) % nf
            im = m % ni
            slot = m % 2
            for j in range(cb):
                co = coef_ref[bm, im * cb + j]

                @pl.when(co != 0.0)
                def _():
                    pltpu.make_async_copy(
                        x_hbm.at[bm, fm, im * cb + j],
                        buf.at[slot, j],
                        sems.at[slot, j],
                    ).start()

        @pl.when(n == 0)
        def _():
            issue(n)

        @pl.when(n + 1 < total)
        def _():
            issue(n + 1)

        slot = n % 2
        for j in range(cb):
            co = coef_ref[b, i * cb + j]

            @pl.when(co != 0.0)
            def _():
                pltpu.make_async_copy(
                    x_hbm.at[b, f, i * cb + j],
                    buf.at[slot, j],
                    sems.at[slot, j],
                ).wait()
                o_ref[0, 0, j] = buf[slot, j] * co

            @pl.when(co == 0.0)
            def _():
                o_ref[0, 0, j] = jnp.zeros((h, w), jnp.float32)
    return _mul_body


def kernel(x, W1, W2):
    B, F, C, H, W = x.shape
    k = int(math.ceil(C * C_SPARSITY))

    CB1 = 16
    NC1 = C // CB1
    sums, maxs = pl.pallas_call(
        _pool_body,
        grid=(B, NC1),
        in_specs=[pl.BlockSpec((1, F, CB1, H, W),
                               lambda b, i: (b, 0, i, 0, 0))],
        out_specs=[pl.BlockSpec((1, 1, 1, CB1), lambda b, i: (b, i, 0, 0)),
                   pl.BlockSpec((1, 1, 1, CB1), lambda b, i: (b, i, 0, 0))],
        out_shape=[jax.ShapeDtypeStruct((B, NC1, 1, CB1), jnp.float32),
                   jax.ShapeDtypeStruct((B, NC1, 1, CB1), jnp.float32)],
    )(x)
    sums = sums.reshape(B, C)
    maxs = maxs.reshape(B, C)

    coef = pl.pallas_call(
        _make_mask_body(float(F * H * W), k),
        out_shape=jax.ShapeDtypeStruct((B, C), jnp.float32),
    )(sums, maxs, W1, W2)

    CB3 = 16
    out = pl.pallas_call(
        _make_mul_body(CB3, (H, W)),
        grid=(B, F, C // CB3),
        in_specs=[pl.BlockSpec(memory_space=pltpu.MemorySpace.HBM),
                  pl.BlockSpec(memory_space=pltpu.SMEM)],
        out_specs=pl.BlockSpec((1, 1, CB3, H, W),
                               lambda b, f, i: (b, f, i, 0, 0)),
        out_shape=jax.ShapeDtypeStruct(x.shape, x.dtype),
        scratch_shapes=[pltpu.VMEM((2, CB3, H, W), jnp.float32),
                        pltpu.SemaphoreType.DMA((2, CB3))],
    )(x, coef)
    return out


# speedup vs baseline: 3.4013x; 1.0125x over previous
import math

import jax
import jax.numpy as jnp
from jax.experimental import pallas as pl
from jax.experimental.pallas import tpu as pltpu

C_SPARSITY = 0.8


def _pool_body(x_ref, sum_ref, max_ref):
    xb = x_ref[...]
    s = jnp.sum(xb, axis=(3, 4))
    m = jnp.max(xb, axis=(3, 4))
    sum_ref[0, 0, 0, :] = jnp.sum(s, axis=(0, 1))
    max_ref[0, 0, 0, :] = jnp.max(m, axis=(0, 1))


def _make_mask_body(n_pool, k):
    def _mask_body(sum_ref, max_ref, w1_ref, w2_ref, coef_ref):
        avg = sum_ref[...] * (1.0 / n_pool)
        mx = max_ref[...]
        w1 = w1_ref[...]
        w2 = w2_ref[...]

        def mlp(v):
            h = jax.lax.dot_general(v, w1, (((1,), (1,)), ((), ())),
                                    preferred_element_type=jnp.float32)
            h = jnp.maximum(h, 0.0)
            return jax.lax.dot_general(h, w2, (((1,), (1,)), ((), ())),
                                       preferred_element_type=jnp.float32)

        s = jax.nn.sigmoid(mlp(avg) + mlp(mx))
        c = s.shape[1]
        si = s[:, :, None]
        sj = s[:, None, :]
        gt = jnp.sum(jnp.where(sj > si, 1.0, 0.0), axis=2)
        ii = jax.lax.broadcasted_iota(jnp.int32, (1, c, c), 1)
        jj = jax.lax.broadcasted_iota(jnp.int32, (1, c, c), 2)
        eq_lower = jnp.sum(
            jnp.where((sj == si) & (jj < ii), 1.0, 0.0), axis=2)
        rank = gt + eq_lower
        mask = jnp.where(rank < float(k), 1.0, 0.0)
        coef_ref[...] = s * mask
    return _mask_body


def _make_mul_body(cb, hw):
    h, w = hw

    def _mul_body(x_hbm, coef_ref, o_ref, buf, sems):
        b = pl.program_id(0)
        f = pl.program_id(1)
        i = pl.program_id(2)
        nf = pl.num_programs(1)
        ni = pl.num_programs(2)
        n = (b * nf + f) * ni + i
        total = pl.num_programs(0) * nf * ni

        def issue(m):
            bm = m // (nf * ni)
            fm = (m // ni) % nf
            im = m % ni
            slot = m % 2
            for j in range(cb):
                co = coef_ref[bm, im * cb + j]

                @pl.when(co != 0.0)
                def _():
                    pltpu.make_async_copy(
                        x_hbm.at[bm, fm, im * cb + j],
                        buf.at[slot, j],
                        sems.at[slot, j],
                    ).start()

        @pl.when(n == 0)
        def _():
            issue(n)

        @pl.when(n + 1 < total)
        def _():
            issue(n + 1)

        slot = n % 2
        for j in range(cb):
            co = coef_ref[b, i * cb + j]

            @pl.when(co != 0.0)
            def _():
                pltpu.make_async_copy(
                    x_hbm.at[b, f, i * cb + j],
                    buf.at[slot, j],
                    sems.at[slot, j],
                ).wait()
                o_ref[0, 0, j] = buf[slot, j] * co

            @pl.when(co == 0.0)
            def _():
                o_ref[0, 0, j] = jnp.zeros((h, w), jnp.float32)
    return _mul_body


def kernel(x, W1, W2):
    B, F, C, H, W = x.shape
    k = int(math.ceil(C * C_SPARSITY))

    CB1 = 16
    NC1 = C // CB1
    sums, maxs = pl.pallas_call(
        _pool_body,
        grid=(B, NC1),
        in_specs=[pl.BlockSpec((1, F, CB1, H, W),
                               lambda b, i: (b, 0, i, 0, 0))],
        out_specs=[pl.BlockSpec((1, 1, 1, CB1), lambda b, i: (b, i, 0, 0)),
                   pl.BlockSpec((1, 1, 1, CB1), lambda b, i: (b, i, 0, 0))],
        out_shape=[jax.ShapeDtypeStruct((B, NC1, 1, CB1), jnp.float32),
                   jax.ShapeDtypeStruct((B, NC1, 1, CB1), jnp.float32)],
    )(x)
    sums = sums.reshape(B, C)
    maxs = maxs.reshape(B, C)

    coef = pl.pallas_call(
        _make_mask_body(float(F * H * W), k),
        out_shape=jax.ShapeDtypeStruct((B, C), jnp.float32),
    )(sums, maxs, W1, W2)

    CB3 = 32
    out = pl.pallas_call(
        _make_mul_body(CB3, (H, W)),
        grid=(B, F, C // CB3),
        in_specs=[pl.BlockSpec(memory_space=pltpu.MemorySpace.HBM),
                  pl.BlockSpec(memory_space=pltpu.SMEM)],
        out_specs=pl.BlockSpec((1, 1, CB3, H, W),
                               lambda b, f, i: (b, f, i, 0, 0)),
        out_shape=jax.ShapeDtypeStruct(x.shape, x.dtype),
        scratch_shapes=[pltpu.VMEM((2, CB3, H, W), jnp.float32),
                        pltpu.SemaphoreType.DMA((2, CB3))],
    )(x, coef)
    return out
